# Initial kernel scaffold; baseline (speedup 1.0000x reference)
import functools

import jax
import jax.numpy as jnp
from jax import lax
from jax.experimental import pallas as pl
from jax.experimental.pallas import tpu as pltpu
from jax.experimental.pallas import tpu_sc as plsc

N = 10000
E = 320000
F = 128
NC = 2
NS = 16
NW = NC * NS
EPT = E // NW
B = 80
STEPS = EPT // B
RPW = N // NS

_vmesh = plsc.VectorSubcoreMesh(core_axis_name="c", subcore_axis_name="s")


def _deg_body(dst_hbm, z1_hbm, out_hbm, idx_v, ones_v, acc_sh):
    cid = lax.axis_index("c")
    sid = lax.axis_index("s")
    base = (cid * NS + sid) * EPT

    @pl.when(sid == 0)
    def _():
        pltpu.sync_copy(z1_hbm, acc_sh)

    @pl.loop(0, B, step=16)
    def _(i):
        ones_v[pl.ds(i, 16)] = jnp.full((16,), 1.0, jnp.float32)

    plsc.subcore_barrier()

    @pl.loop(0, STEPS)
    def _(j):
        pltpu.sync_copy(dst_hbm.at[pl.ds(base + j * B, B)], idx_v)
        pltpu.sync_copy(ones_v, acc_sh.at[idx_v], add=True)

    plsc.subcore_barrier()

    @pl.when(sid == 0)
    def _():
        pltpu.sync_copy(acc_sh, out_hbm.at[pl.ds(cid * N, N)])


_deg_call = functools.partial(
    pl.kernel,
    out_type=jax.ShapeDtypeStruct((NC * N,), jnp.float32),
    mesh=_vmesh,
    scratch_types=[
        pltpu.VMEM((B,), jnp.int32),
        pltpu.VMEM((B,), jnp.float32),
        pltpu.VMEM_SHARED((N,), jnp.float32),
    ],
)(_deg_body)


def _agg_body(src_hbm, dst_hbm, m_hbm, z2_hbm, out_hbm,
              idxs_v, idxd_v, rows_v, acc_sh):
    cid = lax.axis_index("c")
    sid = lax.axis_index("s")
    base = (cid * NS + sid) * EPT

    pltpu.sync_copy(z2_hbm.at[pl.ds(sid * RPW, RPW)],
                    acc_sh.at[pl.ds(sid * RPW, RPW)])
    plsc.subcore_barrier()

    @pl.loop(0, STEPS)
    def _(j):
        e0 = base + j * B
        pltpu.sync_copy(src_hbm.at[pl.ds(e0, B)], idxs_v)
        pltpu.sync_copy(dst_hbm.at[pl.ds(e0, B)], idxd_v)
        pltpu.sync_copy(m_hbm.at[idxs_v], rows_v)
        pltpu.sync_copy(rows_v, acc_sh.at[idxd_v], add=True)

    plsc.subcore_barrier()
    pltpu.sync_copy(acc_sh.at[pl.ds(sid * RPW, RPW)],
                    out_hbm.at[pl.ds(cid * N + sid * RPW, RPW)])


_agg_call = functools.partial(
    pl.kernel,
    out_type=jax.ShapeDtypeStruct((NC * N, F), jnp.float32),
    mesh=_vmesh,
    scratch_types=[
        pltpu.VMEM((B,), jnp.int32),
        pltpu.VMEM((B,), jnp.int32),
        pltpu.VMEM((B, F), jnp.float32),
        pltpu.VMEM_SHARED((N, F), jnp.float32),
    ],
)(_agg_body)


_RB = 1000
_GRID = N // _RB


def _mm_body(x_ref, w_ref, o_ref):
    o_ref[...] = jnp.dot(x_ref[...], w_ref[...],
                         preferred_element_type=jnp.float32)


def _tc_matmul(x, w):
    return pl.pallas_call(
        _mm_body,
        grid=(_GRID,),
        in_specs=[pl.BlockSpec((_RB, F), lambda i: (i, 0)),
                  pl.BlockSpec((F, F), lambda i: (0, 0))],
        out_specs=pl.BlockSpec((_RB, F), lambda i: (i, 0)),
        out_shape=jax.ShapeDtypeStruct((N, F), jnp.float32),
    )(x, w)


def _scale_body(degp_ref, xw_ref, xs_ref, dis_ref):
    deg = degp_ref[0] + degp_ref[1] + 1.0
    dis = lax.rsqrt(deg)
    dis_ref[...] = dis
    xs_ref[...] = xw_ref[...] * dis


def _tc_scale(degp, xw):
    return pl.pallas_call(
        _scale_body,
        grid=(_GRID,),
        in_specs=[pl.BlockSpec((NC, _RB, 1), lambda i: (0, i, 0)),
                  pl.BlockSpec((_RB, F), lambda i: (i, 0))],
        out_specs=[pl.BlockSpec((_RB, F), lambda i: (i, 0)),
                   pl.BlockSpec((_RB, 1), lambda i: (i, 0))],
        out_shape=[jax.ShapeDtypeStruct((N, F), jnp.float32),
                   jax.ShapeDtypeStruct((N, 1), jnp.float32)],
    )(degp, xw)


def _relu_body(tp_ref, xs_ref, dis_ref, b_ref, hs_ref):
    dis = dis_ref[...]
    z = (tp_ref[0] + tp_ref[1] + xs_ref[...]) * dis + b_ref[...]
    hs_ref[...] = jnp.maximum(z, 0.0) * dis


def _tc_relu_scale(tp, xs, dis, b):
    return pl.pallas_call(
        _relu_body,
        grid=(_GRID,),
        in_specs=[pl.BlockSpec((NC, _RB, F), lambda i: (0, i, 0)),
                  pl.BlockSpec((_RB, F), lambda i: (i, 0)),
                  pl.BlockSpec((_RB, 1), lambda i: (i, 0)),
                  pl.BlockSpec((1, F), lambda i: (0, 0))],
        out_specs=pl.BlockSpec((_RB, F), lambda i: (i, 0)),
        out_shape=jax.ShapeDtypeStruct((N, F), jnp.float32),
    )(tp, xs, dis, b)


def _head_body(tp_ref, hs_ref, dis_ref, w_ref, b_ref, o_ref):
    p = (tp_ref[0] + tp_ref[1] + hs_ref[...]) * dis_ref[...]
    o_ref[...] = jnp.dot(p, w_ref[...],
                         preferred_element_type=jnp.float32) + b_ref[...]


def _tc_heads(tp, hs, dis, w, b):
    return pl.pallas_call(
        _head_body,
        grid=(_GRID,),
        in_specs=[pl.BlockSpec((NC, _RB, F), lambda i: (0, i, 0)),
                  pl.BlockSpec((_RB, F), lambda i: (i, 0)),
                  pl.BlockSpec((_RB, 1), lambda i: (i, 0)),
                  pl.BlockSpec((F, F), lambda i: (0, 0)),
                  pl.BlockSpec((1, F), lambda i: (0, 0))],
        out_specs=pl.BlockSpec((_RB, F), lambda i: (i, 0)),
        out_shape=jax.ShapeDtypeStruct((N, F), jnp.float32),
    )(tp, hs, dis, w, b)


def kernel(X, edge_index, W1, b1, Wmu, bmu, Wlv, blv):
    src = edge_index[0]
    dst = edge_index[1]
    z1 = jnp.zeros((N,), jnp.float32)
    z2 = jnp.zeros((N, F), jnp.float32)

    degp = _deg_call(dst, z1)
    xw = _tc_matmul(X, W1)
    xs, dis = _tc_scale(degp.reshape(NC, N, 1), xw)

    t1 = _agg_call(src, dst, xs, z2)
    hs = _tc_relu_scale(t1.reshape(NC, N, F), xs, dis, b1.reshape(1, F))

    t2 = _agg_call(src, dst, hs, z2)
    wcat = jnp.concatenate([Wmu, Wlv], axis=1)
    bcat = jnp.concatenate([bmu, blv]).reshape(1, F)
    out = _tc_heads(t2.reshape(NC, N, F), hs, dis, wcat, bcat)

    return out[:, :64], out[:, 64:]


# speedup vs baseline: 14.6280x; 14.6280x over previous
import functools

import jax
import jax.numpy as jnp
from jax import lax
from jax.experimental import pallas as pl
from jax.experimental.pallas import tpu as pltpu
from jax.experimental.pallas import tpu_sc as plsc

NREAL = 10000
N = 10240
E = 320000
F = 128
NC = 2
NS = 16
NW = NC * NS
EPT = E // NW
B = 80
STEPS = EPT // B
RPW = N // NS

_vmesh = plsc.VectorSubcoreMesh(core_axis_name="c", subcore_axis_name="s")


DW = 128


def _deg_body(dst_hbm, z1_hbm, ones_hbm, out_hbm, idx_v, ones_v, acc_sh):
    cid = lax.axis_index("c")
    sid = lax.axis_index("s")
    base = (cid * NS + sid) * EPT

    pltpu.sync_copy(z1_hbm.at[pl.ds(sid * RPW, RPW)],
                    acc_sh.at[pl.ds(sid * RPW, RPW)])
    pltpu.sync_copy(ones_hbm, ones_v)
    plsc.subcore_barrier()

    @pl.loop(0, STEPS)
    def _(j):
        pltpu.sync_copy(dst_hbm.at[pl.ds(base + j * B, B)], idx_v)
        pltpu.sync_copy(ones_v, acc_sh.at[idx_v], add=True)

    plsc.subcore_barrier()
    pltpu.sync_copy(acc_sh.at[pl.ds(sid * RPW, RPW)],
                    out_hbm.at[pl.ds(cid * N + sid * RPW, RPW)])


_deg_call = functools.partial(
    pl.kernel,
    out_type=jax.ShapeDtypeStruct((NC * N, DW), jnp.float32),
    mesh=_vmesh,
    scratch_types=[
        pltpu.VMEM((B,), jnp.int32),
        pltpu.VMEM((B, DW), jnp.float32),
        pltpu.VMEM_SHARED((N, DW), jnp.float32),
    ],
)(_deg_body)


def _agg_body(src_hbm, dst_hbm, m_hbm, z2_hbm, out_hbm,
              idxs_v, idxd_v, rows_v, acc_sh):
    cid = lax.axis_index("c")
    sid = lax.axis_index("s")
    base = (cid * NS + sid) * EPT

    pltpu.sync_copy(z2_hbm.at[pl.ds(sid * RPW, RPW)],
                    acc_sh.at[pl.ds(sid * RPW, RPW)])
    plsc.subcore_barrier()

    @pl.loop(0, STEPS)
    def _(j):
        e0 = base + j * B
        pltpu.sync_copy(src_hbm.at[pl.ds(e0, B)], idxs_v)
        pltpu.sync_copy(dst_hbm.at[pl.ds(e0, B)], idxd_v)
        pltpu.sync_copy(m_hbm.at[idxs_v], rows_v)
        pltpu.sync_copy(rows_v, acc_sh.at[idxd_v], add=True)

    plsc.subcore_barrier()
    pltpu.sync_copy(acc_sh.at[pl.ds(sid * RPW, RPW)],
                    out_hbm.at[pl.ds(cid * N + sid * RPW, RPW)])


_agg_call = functools.partial(
    pl.kernel,
    out_type=jax.ShapeDtypeStruct((NC * N, F), jnp.float32),
    mesh=_vmesh,
    scratch_types=[
        pltpu.VMEM((B,), jnp.int32),
        pltpu.VMEM((B,), jnp.int32),
        pltpu.VMEM((B, F), jnp.float32),
        pltpu.VMEM_SHARED((N, F), jnp.float32),
    ],
)(_agg_body)


_RB = 1024
_GRID = N // _RB


def _mm_body(x_ref, w_ref, o_ref):
    o_ref[...] = jnp.dot(x_ref[...], w_ref[...],
                         preferred_element_type=jnp.float32)


def _tc_matmul(x, w):
    return pl.pallas_call(
        _mm_body,
        grid=(_GRID,),
        in_specs=[pl.BlockSpec((_RB, F), lambda i: (i, 0)),
                  pl.BlockSpec((F, F), lambda i: (0, 0))],
        out_specs=pl.BlockSpec((_RB, F), lambda i: (i, 0)),
        out_shape=jax.ShapeDtypeStruct((N, F), jnp.float32),
    )(x, w)


def _scale_body(degp_ref, xw_ref, xs_ref, dis_ref):
    d = degp_ref[0] + degp_ref[1]
    deg = d[:, 0:1] + 1.0
    dis = lax.rsqrt(deg)
    dis_ref[...] = dis
    xs_ref[...] = xw_ref[...] * dis


def _tc_scale(degp, xw):
    return pl.pallas_call(
        _scale_body,
        grid=(_GRID,),
        in_specs=[pl.BlockSpec((NC, _RB, DW), lambda i: (0, i, 0)),
                  pl.BlockSpec((_RB, F), lambda i: (i, 0))],
        out_specs=[pl.BlockSpec((_RB, F), lambda i: (i, 0)),
                   pl.BlockSpec((_RB, 1), lambda i: (i, 0))],
        out_shape=[jax.ShapeDtypeStruct((N, F), jnp.float32),
                   jax.ShapeDtypeStruct((N, 1), jnp.float32)],
    )(degp, xw)


def _relu_body(tp_ref, xs_ref, dis_ref, b_ref, hs_ref):
    dis = dis_ref[...]
    z = (tp_ref[0] + tp_ref[1] + xs_ref[...]) * dis + b_ref[...]
    hs_ref[...] = jnp.maximum(z, 0.0) * dis


def _tc_relu_scale(tp, xs, dis, b):
    return pl.pallas_call(
        _relu_body,
        grid=(_GRID,),
        in_specs=[pl.BlockSpec((NC, _RB, F), lambda i: (0, i, 0)),
                  pl.BlockSpec((_RB, F), lambda i: (i, 0)),
                  pl.BlockSpec((_RB, 1), lambda i: (i, 0)),
                  pl.BlockSpec((1, F), lambda i: (0, 0))],
        out_specs=pl.BlockSpec((_RB, F), lambda i: (i, 0)),
        out_shape=jax.ShapeDtypeStruct((N, F), jnp.float32),
    )(tp, xs, dis, b)


def _head_body(tp_ref, hs_ref, dis_ref, w_ref, b_ref, o_ref):
    p = (tp_ref[0] + tp_ref[1] + hs_ref[...]) * dis_ref[...]
    o_ref[...] = jnp.dot(p, w_ref[...],
                         preferred_element_type=jnp.float32) + b_ref[...]


def _tc_heads(tp, hs, dis, w, b):
    return pl.pallas_call(
        _head_body,
        grid=(_GRID,),
        in_specs=[pl.BlockSpec((NC, _RB, F), lambda i: (0, i, 0)),
                  pl.BlockSpec((_RB, F), lambda i: (i, 0)),
                  pl.BlockSpec((_RB, 1), lambda i: (i, 0)),
                  pl.BlockSpec((F, F), lambda i: (0, 0)),
                  pl.BlockSpec((1, F), lambda i: (0, 0))],
        out_specs=pl.BlockSpec((_RB, F), lambda i: (i, 0)),
        out_shape=jax.ShapeDtypeStruct((N, F), jnp.float32),
    )(tp, hs, dis, w, b)


def kernel(X, edge_index, W1, b1, Wmu, bmu, Wlv, blv):
    src = edge_index[0]
    dst = edge_index[1]
    Xp = jnp.pad(X, ((0, N - NREAL), (0, 0)))
    z2 = jnp.zeros((N, F), jnp.float32)

    ones = jnp.ones((B, DW), jnp.float32)
    degp = _deg_call(dst, z2, ones)
    xw = _tc_matmul(Xp, W1)
    xs, dis = _tc_scale(degp.reshape(NC, N, DW), xw)

    t1 = _agg_call(src, dst, xs, z2)
    hs = _tc_relu_scale(t1.reshape(NC, N, F), xs, dis, b1.reshape(1, F))

    t2 = _agg_call(src, dst, hs, z2)
    wcat = jnp.concatenate([Wmu, Wlv], axis=1)
    bcat = jnp.concatenate([bmu, blv]).reshape(1, F)
    out = _tc_heads(t2.reshape(NC, N, F), hs, dis, wcat, bcat)

    return out[:NREAL, :64], out[:NREAL, 64:]


# speedup vs baseline: 25.6819x; 1.7557x over previous
import functools

import jax
import jax.numpy as jnp
from jax import lax
from jax.experimental import pallas as pl
from jax.experimental.pallas import tpu as pltpu
from jax.experimental.pallas import tpu_sc as plsc

NREAL = 10000
N = 10240
E = 320000
F = 128
NC = 2
NS = 16
NW = NC * NS
EPT = E // NW
B = 80
STEPS = EPT // B
NBUF = 2
RPW = N // NS

_vmesh = plsc.VectorSubcoreMesh(core_axis_name="c", subcore_axis_name="s")


DW = 128


def _deg_body(dsti_hbm, z_hbm, ones_hbm, out_hbm, idx_v, ones_v, acc_sh,
              s0, s1, s2, s3):
    cid = lax.axis_index("c")
    sid = lax.axis_index("s")
    wid = cid * NS + sid
    ssem = [s0, s1, s2, s3]

    pltpu.sync_copy(z_hbm.at[pl.ds(sid * RPW, RPW)],
                    acc_sh.at[pl.ds(sid * RPW, RPW)])
    pltpu.sync_copy(ones_hbm, ones_v)
    pltpu.sync_copy(dsti_hbm.at[wid], idx_v)
    plsc.subcore_barrier()

    @pl.loop(0, STEPS - 1, step=4)
    def _(g):
        for b in range(4):
            t = g + b

            @pl.when(t >= 4)
            def _():
                pltpu.make_async_copy(
                    ones_v, acc_sh.at[idx_v.at[t - 4]], ssem[b]).wait()

            pltpu.async_copy(ones_v, acc_sh.at[idx_v.at[t]], ssem[b],
                             add=True)

    pltpu.make_async_copy(
        ones_v, acc_sh.at[idx_v.at[STEPS - 5]], ssem[0]).wait()
    pltpu.async_copy(ones_v, acc_sh.at[idx_v.at[STEPS - 1]], ssem[0],
                     add=True)
    for t in range(STEPS - 4, STEPS):
        pltpu.make_async_copy(
            ones_v, acc_sh.at[idx_v.at[t]], ssem[t % 4]).wait()

    plsc.subcore_barrier()
    pltpu.sync_copy(acc_sh.at[pl.ds(sid * RPW, RPW)],
                    out_hbm.at[pl.ds(cid * N + sid * RPW, RPW)])


_deg_call = functools.partial(
    pl.kernel,
    out_type=jax.ShapeDtypeStruct((NC * N, DW), jnp.float32),
    mesh=_vmesh,
    scratch_types=[
        pltpu.VMEM((STEPS, B), jnp.int32),
        pltpu.VMEM((B, DW), jnp.float32),
        pltpu.VMEM_SHARED((N, DW), jnp.float32),
        pltpu.SemaphoreType.DMA,
        pltpu.SemaphoreType.DMA,
        pltpu.SemaphoreType.DMA,
        pltpu.SemaphoreType.DMA,
    ],
)(_deg_body)


def _agg_body(srci_hbm, dsti_hbm, m_hbm, z_hbm, out_hbm,
              idxs_v, idxd_v, r0, r1, acc_sh, g0, g1, s0, s1):
    cid = lax.axis_index("c")
    sid = lax.axis_index("s")
    wid = cid * NS + sid
    rows = [r0, r1]
    gsem = [g0, g1]
    ssem = [s0, s1]

    pltpu.sync_copy(z_hbm.at[pl.ds(sid * RPW, RPW)],
                    acc_sh.at[pl.ds(sid * RPW, RPW)])
    pltpu.sync_copy(srci_hbm.at[wid], idxs_v)
    pltpu.sync_copy(dsti_hbm.at[wid], idxd_v)
    plsc.subcore_barrier()

    pltpu.async_copy(m_hbm.at[idxs_v.at[pl.ds(0, B)]], rows[0], gsem[0])

    @pl.loop(0, STEPS - 1, step=2)
    def _(g):
        for b in range(2):
            t = g + b
            o = 1 - b

            pltpu.make_async_copy(
                m_hbm.at[idxs_v.at[pl.ds(t * B, B)]], rows[b],
                gsem[b]).wait()
            pltpu.async_copy(rows[b], acc_sh.at[idxd_v.at[t]], ssem[b],
                             add=True)

            @pl.when(t >= 1)
            def _():
                pltpu.make_async_copy(
                    rows[o], acc_sh.at[idxd_v.at[t - 1]],
                    ssem[o]).wait()
            pltpu.async_copy(
                m_hbm.at[idxs_v.at[pl.ds((t + 1) * B, B)]], rows[o],
                gsem[o])

    pltpu.make_async_copy(
        m_hbm.at[idxs_v.at[pl.ds((STEPS - 1) * B, B)]], rows[0],
        gsem[0]).wait()
    pltpu.async_copy(rows[0], acc_sh.at[idxd_v.at[STEPS - 1]], ssem[0],
                     add=True)
    pltpu.make_async_copy(
        rows[1], acc_sh.at[idxd_v.at[STEPS - 2]], ssem[1]).wait()
    pltpu.make_async_copy(
        rows[0], acc_sh.at[idxd_v.at[STEPS - 1]], ssem[0]).wait()

    plsc.subcore_barrier()
    pltpu.sync_copy(acc_sh.at[pl.ds(sid * RPW, RPW)],
                    out_hbm.at[pl.ds(cid * N + sid * RPW, RPW)])


_agg_call = functools.partial(
    pl.kernel,
    out_type=jax.ShapeDtypeStruct((NC * N, F), jnp.float32),
    mesh=_vmesh,
    scratch_types=[
        pltpu.VMEM((EPT,), jnp.int32),
        pltpu.VMEM((STEPS, B), jnp.int32),
        pltpu.VMEM((B, F), jnp.float32),
        pltpu.VMEM((B, F), jnp.float32),
        pltpu.VMEM_SHARED((N, F), jnp.float32),
        pltpu.SemaphoreType.DMA,
        pltpu.SemaphoreType.DMA,
        pltpu.SemaphoreType.DMA,
        pltpu.SemaphoreType.DMA,
    ],
)(_agg_body)


_RB = 1024
_GRID = N // _RB


def _mm_body(x_ref, w_ref, o_ref):
    o_ref[...] = jnp.dot(x_ref[...], w_ref[...],
                         preferred_element_type=jnp.float32)


def _tc_matmul(x, w):
    return pl.pallas_call(
        _mm_body,
        grid=(_GRID,),
        in_specs=[pl.BlockSpec((_RB, F), lambda i: (i, 0)),
                  pl.BlockSpec((F, F), lambda i: (0, 0))],
        out_specs=pl.BlockSpec((_RB, F), lambda i: (i, 0)),
        out_shape=jax.ShapeDtypeStruct((N, F), jnp.float32),
    )(x, w)


def _scale_body(degp_ref, xw_ref, xs_ref, dis_ref):
    d = degp_ref[0] + degp_ref[1]
    deg = d[:, 0:1] + 1.0
    dis = lax.rsqrt(deg)
    dis_ref[...] = dis
    xs_ref[...] = xw_ref[...] * dis


def _tc_scale(degp, xw):
    return pl.pallas_call(
        _scale_body,
        grid=(_GRID,),
        in_specs=[pl.BlockSpec((NC, _RB, DW), lambda i: (0, i, 0)),
                  pl.BlockSpec((_RB, F), lambda i: (i, 0))],
        out_specs=[pl.BlockSpec((_RB, F), lambda i: (i, 0)),
                   pl.BlockSpec((_RB, 1), lambda i: (i, 0))],
        out_shape=[jax.ShapeDtypeStruct((N, F), jnp.float32),
                   jax.ShapeDtypeStruct((N, 1), jnp.float32)],
    )(degp, xw)


def _relu_body(tp_ref, xs_ref, dis_ref, b_ref, hs_ref):
    dis = dis_ref[...]
    z = (tp_ref[0] + tp_ref[1] + xs_ref[...]) * dis + b_ref[...]
    hs_ref[...] = jnp.maximum(z, 0.0) * dis


def _tc_relu_scale(tp, xs, dis, b):
    return pl.pallas_call(
        _relu_body,
        grid=(_GRID,),
        in_specs=[pl.BlockSpec((NC, _RB, F), lambda i: (0, i, 0)),
                  pl.BlockSpec((_RB, F), lambda i: (i, 0)),
                  pl.BlockSpec((_RB, 1), lambda i: (i, 0)),
                  pl.BlockSpec((1, F), lambda i: (0, 0))],
        out_specs=pl.BlockSpec((_RB, F), lambda i: (i, 0)),
        out_shape=jax.ShapeDtypeStruct((N, F), jnp.float32),
    )(tp, xs, dis, b)


def _head_body(tp_ref, hs_ref, dis_ref, w_ref, b_ref, o_ref):
    p = (tp_ref[0] + tp_ref[1] + hs_ref[...]) * dis_ref[...]
    o_ref[...] = jnp.dot(p, w_ref[...],
                         preferred_element_type=jnp.float32) + b_ref[...]


def _tc_heads(tp, hs, dis, w, b):
    return pl.pallas_call(
        _head_body,
        grid=(_GRID,),
        in_specs=[pl.BlockSpec((NC, _RB, F), lambda i: (0, i, 0)),
                  pl.BlockSpec((_RB, F), lambda i: (i, 0)),
                  pl.BlockSpec((_RB, 1), lambda i: (i, 0)),
                  pl.BlockSpec((F, F), lambda i: (0, 0)),
                  pl.BlockSpec((1, F), lambda i: (0, 0))],
        out_specs=pl.BlockSpec((_RB, F), lambda i: (i, 0)),
        out_shape=jax.ShapeDtypeStruct((N, F), jnp.float32),
    )(tp, hs, dis, w, b)


def kernel(X, edge_index, W1, b1, Wmu, bmu, Wlv, blv):
    srci = edge_index[0].reshape(NW, EPT)
    dsti = edge_index[1].reshape(NW, STEPS, B)
    Xp = jnp.pad(X, ((0, N - NREAL), (0, 0)))
    z2 = jnp.zeros((N, F), jnp.float32)
    ones = jnp.ones((B, DW), jnp.float32)

    degp = _deg_call(dsti, z2, ones)
    xw = _tc_matmul(Xp, W1)
    xs, dis = _tc_scale(degp.reshape(NC, N, DW), xw)

    t1 = _agg_call(srci, dsti, xs, z2)
    hs = _tc_relu_scale(t1.reshape(NC, N, F), xs, dis, b1.reshape(1, F))

    t2 = _agg_call(srci, dsti, hs, z2)
    wcat = jnp.concatenate([Wmu, Wlv], axis=1)
    bcat = jnp.concatenate([bmu, blv]).reshape(1, F)
    out = _tc_heads(t2.reshape(NC, N, F), hs, dis, wcat, bcat)

    return out[:NREAL, :64], out[:NREAL, 64:]
